# Initial kernel scaffold; baseline (speedup 1.0000x reference)
import jax
import jax.numpy as jnp
from jax.experimental import pallas as pl


def kernel(points_xyz, points_fts):
    raise NotImplementedError("write your pallas kernel here")


# speedup vs baseline: 8.2961x; 8.2961x over previous
import functools

import jax
import jax.numpy as jnp
from jax import lax
from jax.experimental import pallas as pl
from jax.experimental.pallas import tpu as pltpu
from jax.experimental.pallas import tpu_sc as plsc

_NSAMPLE = 32
_MB = 256
_D_PAD = 128

_NC = 2
_NS = 16
_NW = _NC * _NS
_ROW_W = 128
_CHUNK = 512


def _topk_body(kmat_ref, qt_ref, kk_ref, qq_ref, idx_ref, d_ref):
    kmat = kmat_ref[0]
    qt = qt_ref[0]
    inner = jnp.dot(kmat, qt, preferred_element_type=jnp.float32)
    qq = qq_ref[0]
    kk = kk_ref[0]
    d_ref[...] = (qq - 2.0 * inner) + kk

    n = kmat.shape[0]
    mb = qt.shape[1]
    iota = lax.broadcasted_iota(jnp.int32, (n, mb), 0)

    def body(j, _):
        d = d_ref[...]
        v = jnp.min(d, axis=0)
        hit = d <= v[None, :]
        am = jnp.min(jnp.where(hit, iota, n), axis=0)
        idx_ref[0, j, :] = am
        d_ref[...] = jnp.where(iota == am[None, :], jnp.inf, d)
        return 0

    lax.fori_loop(0, _NSAMPLE, body, 0)


def _knn_topk(kmat, qt, kk, qq):
    b, n, _ = kmat.shape
    grid = (b, n // _MB)
    return pl.pallas_call(
        _topk_body,
        grid=grid,
        in_specs=[
            pl.BlockSpec((1, n, _D_PAD), lambda bi, i: (bi, 0, 0)),
            pl.BlockSpec((1, _D_PAD, _MB), lambda bi, i: (bi, 0, i)),
            pl.BlockSpec((1, n, 1), lambda bi, i: (bi, 0, 0)),
            pl.BlockSpec((1, 1, _MB), lambda bi, i: (bi, 0, i)),
        ],
        out_specs=pl.BlockSpec((1, _NSAMPLE, _MB), lambda bi, i: (bi, 0, i)),
        out_shape=jax.ShapeDtypeStruct((b, _NSAMPLE, n), jnp.int32),
        scratch_shapes=[pltpu.VMEM((n, _MB), jnp.float32)],
    )(kmat, qt, kk, qq)


def _gather_tec(table_hbm, gidx_hbm, out_hbm, idx_v, rows_v, sem):
    wid = lax.axis_index("s") * _NC + lax.axis_index("c")
    rows_total = gidx_hbm.shape[0]
    per_w = rows_total // _NW
    base = wid * per_w

    def chunk(i, _):
        b0 = base + i * _CHUNK
        pltpu.sync_copy(gidx_hbm.at[pl.ds(b0, _CHUNK)], idx_v)
        pltpu.async_copy(table_hbm.at[idx_v], rows_v, sem).wait()
        pltpu.sync_copy(rows_v, out_hbm.at[pl.ds(b0, _CHUNK)])
        return 0

    lax.fori_loop(0, per_w // _CHUNK, chunk, 0)


def _sc_gather(table, gidx):
    rows_total = gidx.shape[0]
    mesh = plsc.VectorSubcoreMesh(core_axis_name="c", subcore_axis_name="s")
    k = pl.kernel(
        _gather_tec,
        out_type=jax.ShapeDtypeStruct((rows_total, _ROW_W), jnp.float32),
        mesh=mesh,
        scratch_types=[
            pltpu.VMEM((_CHUNK,), jnp.int32),
            pltpu.VMEM((_CHUNK, _ROW_W), jnp.float32),
            pltpu.SemaphoreType.DMA,
        ],
    )
    return k(table, gidx)


def kernel(points_xyz, points_fts):
    b, c, _, n = points_fts.shape
    cd = c * 3

    keys = jnp.transpose(points_fts.reshape(b, cd, n), (0, 2, 1))
    kk = jnp.sum(keys * keys, axis=-1)
    qq = kk

    kmat = jnp.concatenate(
        [keys, jnp.zeros((b, n, _D_PAD - cd), jnp.float32)], axis=-1)
    qt = jnp.transpose(kmat, (0, 2, 1))

    idx_t = _knn_topk(kmat, qt, kk.reshape(b, n, 1), qq.reshape(b, 1, n))

    xyz_pad = jnp.concatenate(
        [points_xyz, jnp.zeros((b, n, _ROW_W - cd - 3), jnp.float32)], axis=-1)
    table = jnp.concatenate([keys, xyz_pad], axis=-1).reshape(b * n, _ROW_W)

    gidx = (idx_t + (jnp.arange(b, dtype=jnp.int32) * n)[:, None, None])
    big = _sc_gather(table, gidx.reshape(-1))
    big = big.reshape(b, _NSAMPLE, n, _ROW_W)

    group_fts = jnp.transpose(big[..., :cd], (0, 3, 2, 1)).reshape(
        b, c, 3, n, _NSAMPLE)
    group_xyz = jnp.transpose(big[..., cd:cd + 3], (0, 2, 1, 3))
    new_fts_out = group_fts[..., 0:1]
    return (group_xyz, group_fts, points_xyz, new_fts_out)


# speedup vs baseline: 9.6286x; 1.1606x over previous
import functools

import jax
import jax.numpy as jnp
from jax import lax
from jax.experimental import pallas as pl
from jax.experimental.pallas import tpu as pltpu
from jax.experimental.pallas import tpu_sc as plsc

_NSAMPLE = 32
_MB = 256
_D_PAD = 128
_CD = 96

_NC = 2
_NS = 16
_NW = _NC * _NS
_ROW_W = 128
_CHUNK = 512


def _topk_body(q_ref, kt_ref, idx_ref, d_ref):
    q = q_ref[0]
    kt = kt_ref[0]
    inner = jnp.dot(q, kt, preferred_element_type=jnp.float32)
    ksq = kt * kt
    kk = ksq[0:1, :]
    for i in range(1, _CD):
        kk = kk + ksq[i:i + 1, :]
    qsq = q * q
    qq = qsq[:, 0:1]
    for i in range(1, _CD):
        qq = qq + qsq[:, i:i + 1]
    d_ref[...] = (qq - 2.0 * inner) + kk

    mb, n = inner.shape
    iota = lax.broadcasted_iota(jnp.int32, (mb, n), 1)
    kiota = lax.broadcasted_iota(jnp.int32, (mb, _NSAMPLE), 1)

    def body(j, idxacc):
        d = d_ref[...]
        v = jnp.min(d, axis=1, keepdims=True)
        am = jnp.min(jnp.where(d <= v, iota, n), axis=1,
                     keepdims=True)
        d_ref[...] = jnp.where(iota == am, jnp.inf, d)
        return jnp.where(kiota == j, am, idxacc)

    idx_ref[0] = lax.fori_loop(
        0, _NSAMPLE, body, jnp.zeros((mb, _NSAMPLE), jnp.int32))


def _knn_topk(q_pad, kt):
    b, n, _ = q_pad.shape
    grid = (b, n // _MB)
    return pl.pallas_call(
        _topk_body,
        grid=grid,
        in_specs=[
            pl.BlockSpec((1, _MB, _D_PAD), lambda bi, i: (bi, i, 0)),
            pl.BlockSpec((1, _D_PAD, n), lambda bi, i: (bi, 0, 0)),
        ],
        out_specs=pl.BlockSpec((1, _MB, _NSAMPLE), lambda bi, i: (bi, i, 0)),
        out_shape=jax.ShapeDtypeStruct((b, n, _NSAMPLE), jnp.int32),
        scratch_shapes=[pltpu.VMEM((_MB, n), jnp.float32)],
    )(q_pad, kt)


def _gather_tec(table_hbm, gidx_hbm, out_hbm, idx_v, rows_v, sem):
    wid = lax.axis_index("s") * _NC + lax.axis_index("c")
    rows_total = gidx_hbm.shape[0]
    per_w = rows_total // _NW
    base = wid * per_w

    def chunk(i, _):
        b0 = base + i * _CHUNK
        pltpu.sync_copy(gidx_hbm.at[pl.ds(b0, _CHUNK)], idx_v)
        pltpu.async_copy(table_hbm.at[idx_v], rows_v, sem).wait()
        pltpu.sync_copy(rows_v, out_hbm.at[pl.ds(b0, _CHUNK)])
        return 0

    lax.fori_loop(0, per_w // _CHUNK, chunk, 0)


def _sc_gather(table, gidx):
    rows_total = gidx.shape[0]
    mesh = plsc.VectorSubcoreMesh(core_axis_name="c", subcore_axis_name="s")
    k = pl.kernel(
        _gather_tec,
        out_type=jax.ShapeDtypeStruct((rows_total, _ROW_W), jnp.float32),
        mesh=mesh,
        scratch_types=[
            pltpu.VMEM((_CHUNK,), jnp.int32),
            pltpu.VMEM((_CHUNK, _ROW_W), jnp.float32),
            pltpu.SemaphoreType.DMA,
        ],
    )
    return k(table, gidx)


def kernel(points_xyz, points_fts):
    b, c, _, n = points_fts.shape
    cd = c * 3

    keys = jnp.transpose(points_fts.reshape(b, cd, n), (0, 2, 1))
    kmat = jnp.concatenate(
        [keys, jnp.zeros((b, n, _D_PAD - cd), jnp.float32)], axis=-1)
    qt = jnp.transpose(kmat, (0, 2, 1))

    idx = _knn_topk(kmat, qt)

    xyz_pad = jnp.concatenate(
        [points_xyz, jnp.zeros((b, n, _ROW_W - cd - 3), jnp.float32)], axis=-1)
    table = jnp.concatenate([keys, xyz_pad], axis=-1).reshape(b * n, _ROW_W)

    gidx = (idx + (jnp.arange(b, dtype=jnp.int32) * n)[:, None, None])
    big = _sc_gather(table, gidx.reshape(-1))
    big = big.reshape(b, n, _NSAMPLE, _ROW_W)

    group_fts = jnp.transpose(big[..., :cd], (0, 3, 1, 2)).reshape(
        b, c, 3, n, _NSAMPLE)
    group_xyz = big[..., cd:cd + 3]
    new_fts_out = group_fts[..., 0:1]
    return (group_xyz, group_fts, points_xyz, new_fts_out)
